# baseline (speedup 1.0000x reference)
import functools

import jax
import jax.numpy as jnp
from jax import lax
from jax.experimental import pallas as pl
from jax.experimental.pallas import tpu as pltpu
from jax.experimental.pallas import tpu_sc as plsc

N = 10000
D = 128
E = 320000

NC = 2
NS = 16
NW = NC * NS

CHUNK = 128
N_PAD = 10240
STRIPE = N_PAD // NS
BLKS = STRIPE // CHUNK
PER_W = 10240
N_CHUNKS = PER_W // CHUNK
E_PAD = PER_W * NW


def _make_sc_agg(with_deg):
    mesh = plsc.VectorSubcoreMesh(core_axis_name="c", subcore_axis_name="s")

    def body(*args):
        if with_deg:
            (x_hbm, src_hbm, dst_hbm, zrow_hbm, zs_hbm, ones_hbm,
             part_hbm, degp_hbm,
             acc_sh, deg_sh, src_idx, dst_idx, rows, ones_v, vs, sem) = args
        else:
            (x_hbm, src_hbm, dst_hbm, zrow_hbm,
             part_hbm,
             acc_sh, src_idx, dst_idx, rows, sem) = args
        c = lax.axis_index("c")
        s = lax.axis_index("s")
        wid = s * NC + c

        pltpu.sync_copy(zrow_hbm, rows)
        for k in range(BLKS):
            pltpu.sync_copy(rows, acc_sh.at[pl.ds(s * STRIPE + k * CHUNK, CHUNK), :])
        if with_deg:
            pltpu.sync_copy(zs_hbm, vs)
            pltpu.sync_copy(vs, deg_sh.at[pl.ds(s * STRIPE, STRIPE)])
            pltpu.sync_copy(ones_hbm, ones_v)
        plsc.subcore_barrier()

        def chunk_step(j, carry):
            base = wid * PER_W + j * CHUNK
            pltpu.sync_copy(src_hbm.at[pl.ds(base, CHUNK)], src_idx)
            pltpu.sync_copy(dst_hbm.at[pl.ds(base, CHUNK)], dst_idx)
            pltpu.async_copy(x_hbm.at[src_idx], rows, sem).wait()
            pltpu.sync_copy(rows, acc_sh.at[dst_idx], add=True)
            if with_deg:
                pltpu.sync_copy(ones_v, deg_sh.at[dst_idx], add=True)
            return carry

        lax.fori_loop(0, N_CHUNKS, chunk_step, 0)
        plsc.subcore_barrier()

        for k in range(BLKS):
            pltpu.sync_copy(acc_sh.at[pl.ds(s * STRIPE + k * CHUNK, CHUNK), :], rows)
            pltpu.sync_copy(rows, part_hbm.at[pl.ds(c * N_PAD + s * STRIPE + k * CHUNK, CHUNK), :])
        if with_deg:
            pltpu.sync_copy(deg_sh.at[pl.ds(s * STRIPE, STRIPE)], vs)
            pltpu.sync_copy(vs, degp_hbm.at[pl.ds(c * N_PAD + s * STRIPE, STRIPE)])

    out_type = [jax.ShapeDtypeStruct((NC * N_PAD, D), jnp.float32)]
    scratch = [
        pltpu.VMEM_SHARED((N_PAD, D), jnp.float32),
        pltpu.VMEM((CHUNK,), jnp.int32),
        pltpu.VMEM((CHUNK,), jnp.int32),
        pltpu.VMEM((CHUNK, D), jnp.float32),
        pltpu.SemaphoreType.DMA,
    ]
    if with_deg:
        out_type.append(jax.ShapeDtypeStruct((NC * N_PAD,), jnp.float32))
        scratch.insert(1, pltpu.VMEM_SHARED((N_PAD,), jnp.float32))
        scratch.insert(5, pltpu.VMEM((CHUNK,), jnp.float32))
        scratch.insert(6, pltpu.VMEM((STRIPE,), jnp.float32))
    return functools.partial(
        pl.kernel, mesh=mesh, out_type=tuple(out_type), scratch_types=scratch,
    )(body)


_sc_agg_deg = _make_sc_agg(True)
_sc_agg = _make_sc_agg(False)


def _tc_layer_body(relu, p_ref, dp_ref, x_ref, wl_ref, b_ref, wr_ref, o_ref):
    deg = dp_ref[pl.ds(0, N), :] + dp_ref[pl.ds(N_PAD, N), :]
    psum = p_ref[pl.ds(0, N), :] + p_ref[pl.ds(N_PAD, N), :]
    mean = psum * (1.0 / jnp.maximum(deg, 1.0))
    acc = lax.dot_general(mean, wl_ref[...], (((1,), (1,)), ((), ())),
                          preferred_element_type=jnp.float32)
    acc = acc + b_ref[...]
    acc = acc + lax.dot_general(x_ref[...], wr_ref[...], (((1,), (1,)), ((), ())),
                                preferred_element_type=jnp.float32)
    o_ref[...] = jnp.maximum(acc, 0.0) if relu else acc


def _tc_layer(part, degp, x, W_l, b, W_r, relu):
    return pl.pallas_call(
        functools.partial(_tc_layer_body, relu),
        out_shape=jax.ShapeDtypeStruct((N, D), jnp.float32),
    )(part, degp, x, W_l, b.reshape(1, D), W_r)


def kernel(x, edge_index, W1_l, b1, W1_r, W2_l, b2, W2_r):
    pad = E_PAD - E
    src = jnp.concatenate([edge_index[0], jnp.zeros((pad,), jnp.int32)])
    dst = jnp.concatenate([edge_index[1], jnp.full((pad,), N, jnp.int32)])
    zrow = jnp.zeros((CHUNK, D), jnp.float32)
    zs = jnp.zeros((STRIPE,), jnp.float32)
    ones_b = jnp.ones((CHUNK,), jnp.float32)

    part1, degp = _sc_agg_deg(x, src, dst, zrow, zs, ones_b)
    dp = degp.reshape(NC * N_PAD, 1)
    h = _tc_layer(part1, dp, x, W1_l, b1, W1_r, relu=True)

    (part2,) = _sc_agg(h, src, dst, zrow)
    out = _tc_layer(part2, dp, h, W2_l, b2, W2_r, relu=False)
    return out


# speedup vs baseline: 1.3637x; 1.3637x over previous
import functools

import jax
import jax.numpy as jnp
from jax import lax
from jax.experimental import pallas as pl
from jax.experimental.pallas import tpu as pltpu
from jax.experimental.pallas import tpu_sc as plsc

N = 10000
D = 128
E = 320000

NC = 2
NS = 16
NW = NC * NS

CHUNK = 128
N_PAD = 10240
STRIPE = N_PAD // NS
BLKS = STRIPE // CHUNK
PER_W = 10112
N_CHUNKS = PER_W // CHUNK
E_PAD = PER_W * NW


def _make_sc_agg(with_deg):
    mesh = plsc.VectorSubcoreMesh(core_axis_name="c", subcore_axis_name="s")

    def body(*args):
        if with_deg:
            (x_hbm, src_hbm, dst_hbm, zrow_hbm, zs_hbm, ones_hbm,
             part_hbm, degp_hbm,
             acc_sh, deg_sh, src_idx, dst_idx, rows, ones_v, vs, sem) = args
        else:
            (x_hbm, src_hbm, dst_hbm, zrow_hbm,
             part_hbm,
             acc_sh, src_idx, dst_idx, rows, sem) = args
        c = lax.axis_index("c")
        s = lax.axis_index("s")
        wid = s * NC + c

        pltpu.sync_copy(zrow_hbm, rows)
        for k in range(BLKS):
            pltpu.sync_copy(rows, acc_sh.at[pl.ds(s * STRIPE + k * CHUNK, CHUNK), :])
        if with_deg:
            pltpu.sync_copy(zs_hbm, vs)
            pltpu.sync_copy(vs, deg_sh.at[pl.ds(s * STRIPE, STRIPE)])
            pltpu.sync_copy(ones_hbm, ones_v)
        plsc.subcore_barrier()

        def chunk_step(j, carry):
            base = wid * PER_W + j * CHUNK
            pltpu.sync_copy(src_hbm.at[pl.ds(base, CHUNK)], src_idx)
            pltpu.sync_copy(dst_hbm.at[pl.ds(base, CHUNK)], dst_idx)
            pltpu.async_copy(x_hbm.at[src_idx], rows, sem).wait()
            pltpu.sync_copy(rows, acc_sh.at[dst_idx], add=True)
            if with_deg:
                pltpu.sync_copy(ones_v, deg_sh.at[dst_idx], add=True)
            return carry

        lax.fori_loop(0, N_CHUNKS, chunk_step, 0)
        plsc.subcore_barrier()

        for k in range(BLKS):
            pltpu.sync_copy(acc_sh.at[pl.ds(s * STRIPE + k * CHUNK, CHUNK), :], rows)
            pltpu.sync_copy(rows, part_hbm.at[pl.ds(c * N_PAD + s * STRIPE + k * CHUNK, CHUNK), :])
        if with_deg:
            pltpu.sync_copy(deg_sh.at[pl.ds(s * STRIPE, STRIPE)], vs)
            pltpu.sync_copy(vs, degp_hbm.at[pl.ds(c * N_PAD + s * STRIPE, STRIPE)])

    out_type = [jax.ShapeDtypeStruct((NC * N_PAD, D), jnp.float32)]
    scratch = [
        pltpu.VMEM_SHARED((N_PAD, D), jnp.float32),
        pltpu.VMEM((CHUNK,), jnp.int32),
        pltpu.VMEM((CHUNK,), jnp.int32),
        pltpu.VMEM((CHUNK, D), jnp.float32),
        pltpu.SemaphoreType.DMA,
    ]
    if with_deg:
        out_type.append(jax.ShapeDtypeStruct((NC * N_PAD,), jnp.float32))
        scratch.insert(1, pltpu.VMEM_SHARED((N_PAD,), jnp.float32))
        scratch.insert(5, pltpu.VMEM((CHUNK,), jnp.float32))
        scratch.insert(6, pltpu.VMEM((STRIPE,), jnp.float32))
    return functools.partial(
        pl.kernel, mesh=mesh, out_type=tuple(out_type), scratch_types=scratch,
    )(body)


_sc_agg_deg = _make_sc_agg(True)
_sc_agg = _make_sc_agg(False)


def _tc_layer_body(relu, p_ref, dp_ref, x_ref, wl_ref, b_ref, wr_ref, o_ref):
    deg = dp_ref[0] + dp_ref[1]
    mean = (p_ref[0] + p_ref[1]) * (1.0 / jnp.maximum(deg, 1.0))
    acc = lax.dot_general(mean, wl_ref[...], (((1,), (1,)), ((), ())),
                          preferred_element_type=jnp.float32)
    acc = acc + b_ref[...]
    acc = acc + lax.dot_general(x_ref[...], wr_ref[...], (((1,), (1,)), ((), ())),
                                preferred_element_type=jnp.float32)
    o_ref[...] = jnp.maximum(acc, 0.0) if relu else acc


def _tc_layer(part, degp, x, W_l, b, W_r, relu):
    return pl.pallas_call(
        functools.partial(_tc_layer_body, relu),
        out_shape=jax.ShapeDtypeStruct((N, D), jnp.float32),
    )(part, degp, x, W_l, b.reshape(1, D), W_r)


def kernel(x, edge_index, W1_l, b1, W1_r, W2_l, b2, W2_r):
    pad = E_PAD - E
    src = jnp.concatenate([edge_index[0], jnp.zeros((pad,), jnp.int32)])
    dst = jnp.concatenate([edge_index[1], jnp.full((pad,), N, jnp.int32)])
    zrow = jnp.zeros((CHUNK, D), jnp.float32)
    zs = jnp.zeros((STRIPE,), jnp.float32)
    ones_b = jnp.ones((CHUNK,), jnp.float32)

    part1, degp = _sc_agg_deg(x, src, dst, zrow, zs, ones_b)
    part1 = part1.reshape(NC, N_PAD, D)
    dp = degp.reshape(NC, N_PAD, 1)[:, :N, :]
    h = _tc_layer(part1[:, :N, :], dp, x, W1_l, b1, W1_r, relu=True)

    (part2,) = _sc_agg(h, src, dst, zrow)
    part2 = part2.reshape(NC, N_PAD, D)
    out = _tc_layer(part2[:, :N, :], dp, h, W2_l, b2, W2_r, relu=False)
    return out
